# Initial kernel scaffold; baseline (speedup 1.0000x reference)
import jax
import jax.numpy as jnp
from jax.experimental import pallas as pl


def kernel(A, circ_emb, dis_emb, re_CD):
    raise NotImplementedError("write your pallas kernel here")


# speedup vs baseline: 7.8378x; 7.8378x over previous
import functools

import jax
import jax.numpy as jnp
from jax.experimental import pallas as pl
from jax.experimental.pallas import tpu as pltpu

N_LAYERS = 3


def _gcn_kernel(a_ref, c_ref, d_ref, w_ref, circ_out, dis_out, score_out):
    a = a_ref[:]
    ego_c = c_ref[:]
    ego_d = d_ref[:]

    deg_c = jnp.sum(a, axis=1, keepdims=True)
    deg_d = jnp.sum(a, axis=0, keepdims=True).T
    dc = jnp.where(deg_c > 0, jax.lax.rsqrt(deg_c), 0.0)
    dd = jnp.where(deg_d > 0, jax.lax.rsqrt(deg_d), 0.0)

    def cos_weight(y, ego):
        num = jnp.sum(y * ego, axis=1, keepdims=True)
        ny = jnp.sqrt(jnp.sum(y * y, axis=1, keepdims=True))
        ne = jnp.sqrt(jnp.sum(ego * ego, axis=1, keepdims=True))
        return num / jnp.maximum(ny * ne, 1e-8)

    xc, xd = ego_c, ego_d
    acc_c = jnp.zeros_like(ego_c)
    acc_d = jnp.zeros_like(ego_d)
    for _ in range(N_LAYERS):
        yc = dc * jax.lax.dot(a, dd * xd, preferred_element_type=jnp.float32)
        yd = dd * jax.lax.dot_general(
            a, dc * xc, (((0,), (0,)), ((), ())),
            preferred_element_type=jnp.float32)
        xc = cos_weight(yc, ego_c) * yc
        xd = cos_weight(yd, ego_d) * yd
        acc_c = acc_c + xc
        acc_d = acc_d + xd

    circ_out[:] = acc_c
    dis_out[:] = acc_d
    tmp = jax.lax.dot(acc_c, w_ref[:], preferred_element_type=jnp.float32)
    score_out[:] = jax.lax.dot_general(
        tmp, acc_d, (((1,), (1,)), ((), ())),
        preferred_element_type=jnp.float32)


@functools.partial(jax.jit)
def kernel(A, circ_emb, dis_emb, re_CD):
    C, D = A.shape
    L = circ_emb.shape[1]
    out_shapes = (
        jax.ShapeDtypeStruct((C, L), jnp.float32),
        jax.ShapeDtypeStruct((D, L), jnp.float32),
        jax.ShapeDtypeStruct((C, D), jnp.float32),
    )
    return pl.pallas_call(
        _gcn_kernel,
        out_shape=out_shapes,
        compiler_params=pltpu.CompilerParams(
            vmem_limit_bytes=100 * 1024 * 1024,
        ),
    )(A, circ_emb, dis_emb, re_CD)
